# Initial kernel scaffold; baseline (speedup 1.0000x reference)
import jax
import jax.numpy as jnp
from jax.experimental import pallas as pl


def kernel(input_ids, pooling_mask, edge_indices, node_counts, word_emb, pos_emb, ln_gamma, ln_beta, W_gat, a_src, a_dst, W_out):
    raise NotImplementedError("write your pallas kernel here")


# speedup vs baseline: 41.7385x; 41.7385x over previous
import functools

import jax
import jax.numpy as jnp
from jax import lax
from jax.experimental import pallas as pl
from jax.experimental.pallas import tpu as pltpu
from jax.experimental.pallas import tpu_sc as plsc

B, T, N, E, D, S = 8, 512, 256, 8192, 768, 3
NC, NS = 2, 16
NW = NC * NS
ROWS = B * T
RPW = ROWS // NW
GCH = RPW // 2
ECH = 512

def _sc_gather_hist_body(ids_hbm, edges_hbm, table_hbm, tok_hbm, hist_hbm,
                         idx_v, rows_v, acc_v, ebuf_v, sem):
    wid = lax.axis_index("s") * NC + lax.axis_index("c")
    base = wid * RPW

    for half in range(RPW // GCH):
        off = base + half * GCH
        pltpu.sync_copy(ids_hbm.at[pl.ds(off, GCH)], idx_v)
        pltpu.async_copy(table_hbm.at[idx_v], rows_v, sem).wait()
        pltpu.sync_copy(rows_v, tok_hbm.at[pl.ds(off, GCH)])

    @pl.when(wid < B)
    def _():
        b = wid
        zv = jnp.zeros((16,), jnp.float32)

        def zero_body(i, carry):
            for j in range(16):
                acc_v[pl.ds(i * 256 + j * 16, 16)] = zv
            return carry

        lax.fori_loop(0, (N * N) // 256, zero_body, 0)

        ones = jnp.ones((16,), jnp.float32)

        def chunk_body(ci, carry):
            pltpu.sync_copy(edges_hbm.at[b, 0, pl.ds(ci * ECH, ECH)],
                            ebuf_v.at[0])
            pltpu.sync_copy(edges_hbm.at[b, 1, pl.ds(ci * ECH, ECH)],
                            ebuf_v.at[1])

            def vec_body(i, c2):
                src = ebuf_v[0, pl.ds(i * 16, 16)]
                dst = ebuf_v[1, pl.ds(i * 16, 16)]
                pos = dst * N + src
                plsc.addupdate_scatter(acc_v, [pos], ones)
                return c2

            lax.fori_loop(0, ECH // 16, vec_body, 0)
            return carry

        lax.fori_loop(0, E // ECH, chunk_body, 0)
        pltpu.sync_copy(acc_v, hist_hbm.at[b])


@functools.cache
def _sc_gather_hist():
    mesh = plsc.VectorSubcoreMesh(
        core_axis_name="c", subcore_axis_name="s",
        num_cores=NC, num_subcores=NS)
    return pl.kernel(
        _sc_gather_hist_body,
        out_type=(
            jax.ShapeDtypeStruct((ROWS, D), jnp.float32),
            jax.ShapeDtypeStruct((B, N * N), jnp.float32),
        ),
        mesh=mesh,
        scratch_types=[
            pltpu.VMEM((GCH,), jnp.int32),
            pltpu.VMEM((GCH, D), jnp.float32),
            pltpu.VMEM((N * N,), jnp.float32),
            pltpu.VMEM((2, ECH), jnp.int32),
            pltpu.SemaphoreType.DMA,
        ],
        compiler_params=pltpu.CompilerParams(needs_layout_passes=False),
    )


def _tc_body(pm_ref, tok_ref, pos_ref, gam_ref, bet_ref, wg_ref, asrc_ref,
             adst_ref, hist_ref, cnt_ref, wout_ref, out_ref):
    f32 = jnp.float32
    hi = lax.Precision.HIGHEST
    tokb = tok_ref[0] + pos_ref[:]
    node = jnp.dot(pm_ref[0], tokb, preferred_element_type=f32,
                   precision=hi)
    mu = jnp.mean(node, axis=1, keepdims=True)
    xc = node - mu
    var = jnp.mean(xc * xc, axis=1, keepdims=True)
    node = xc * lax.rsqrt(var + 1e-12) * gam_ref[:] + bet_ref[:]
    wh = jnp.dot(node, wg_ref[:], preferred_element_type=f32,
                 precision=hi)
    s_src = lax.dot_general(asrc_ref[:], wh, (((1,), (1,)), ((), ())),
                            preferred_element_type=f32, precision=hi)
    s_dst = jnp.dot(wh, adst_ref[:], preferred_element_type=f32,
                    precision=hi)
    x = s_dst + s_src
    x = jnp.where(x >= 0, x, 0.2 * x)
    cmat = hist_ref[0]
    xm = jnp.where(cmat > 0, x, -1e30)
    emax = jnp.max(xm, axis=1, keepdims=True)
    emax = jnp.where(emax > -1e29, emax, 0.0)
    p = cmat * jnp.exp(xm - emax)
    den = jnp.sum(p, axis=1, keepdims=True)
    a = p / (den + 1e-16)
    msg = jnp.dot(a, wh, preferred_element_type=f32, precision=hi)
    g = jnp.where(msg > 0, msg, jnp.exp(msg) - 1.0)
    gs = jnp.sum(g, axis=0, keepdims=True)
    avg = gs / cnt_ref[pl.program_id(0), 0]
    out_ref[pl.ds(pl.program_id(0), 1), :] = jnp.dot(
        avg, wout_ref[:], preferred_element_type=f32, precision=hi)


_tc_call = pl.pallas_call(
    _tc_body,
    grid=(B,),
    in_specs=[
        pl.BlockSpec((1, N, T), lambda b: (b, 0, 0)),
        pl.BlockSpec((1, T, D), lambda b: (b, 0, 0)),
        pl.BlockSpec((T, D), lambda b: (0, 0)),
        pl.BlockSpec((1, D), lambda b: (0, 0)),
        pl.BlockSpec((1, D), lambda b: (0, 0)),
        pl.BlockSpec((D, D), lambda b: (0, 0)),
        pl.BlockSpec((1, D), lambda b: (0, 0)),
        pl.BlockSpec((D, 1), lambda b: (0, 0)),
        pl.BlockSpec((1, N, N), lambda b: (b, 0, 0)),
        pl.BlockSpec(memory_space=pltpu.SMEM),
        pl.BlockSpec((D, S), lambda b: (0, 0)),
    ],
    out_specs=pl.BlockSpec((B, S), lambda b: (0, 0)),
    out_shape=jax.ShapeDtypeStruct((B, S), jnp.float32),
    compiler_params=pltpu.CompilerParams(
        dimension_semantics=("arbitrary",)),
)


def kernel(input_ids, pooling_mask, edge_indices, node_counts,
           word_emb, pos_emb, ln_gamma, ln_beta, W_gat, a_src, a_dst, W_out):
    ids = input_ids.reshape(ROWS).astype(jnp.int32)
    edges = edge_indices.astype(jnp.int32)
    tok, hist = _sc_gather_hist()(ids, edges, word_emb)
    cnt = jnp.maximum(node_counts, 1).astype(jnp.float32).reshape(B, 1)
    logits = _tc_call(
        pooling_mask, tok.reshape(B, T, D), pos_emb,
        ln_gamma.reshape(1, D), ln_beta.reshape(1, D), W_gat,
        a_src.reshape(1, D), a_dst.reshape(D, 1),
        hist.reshape(B, N, N), cnt, W_out)
    return logits


# speedup vs baseline: 63.5857x; 1.5234x over previous
import functools

import jax
import jax.numpy as jnp
from jax import lax
from jax.experimental import pallas as pl
from jax.experimental.pallas import tpu as pltpu
from jax.experimental.pallas import tpu_sc as plsc

B, T, N, E, D, S = 8, 512, 256, 8192, 768, 3
NC, NS = 2, 16
NW = NC * NS
ROWS = B * T
RPW = ROWS // NW
GCH = RPW // 2
ECH = 512

def _sc_gather_hist_body(ids_hbm, edges_hbm, table_hbm, tok_hbm, hist_hbm,
                         idx_v, rows_v, acc_v, ebuf_v, sem):
    wid = lax.axis_index("s") * NC + lax.axis_index("c")
    base = wid * RPW

    for half in range(RPW // GCH):
        off = base + half * GCH
        pltpu.sync_copy(ids_hbm.at[pl.ds(off, GCH)], idx_v)
        pltpu.async_copy(table_hbm.at[idx_v], rows_v, sem).wait()
        pltpu.sync_copy(rows_v, tok_hbm.at[pl.ds(off, GCH)])

    @pl.when(wid < B)
    def _():
        b = wid
        zv = jnp.zeros((16,), jnp.float32)

        def zero_body(i, carry):
            for j in range(16):
                acc_v[pl.ds(i * 256 + j * 16, 16)] = zv
            return carry

        lax.fori_loop(0, (N * N) // 256, zero_body, 0)

        ones = jnp.ones((16,), jnp.float32)

        def chunk_body(ci, carry):
            pltpu.sync_copy(edges_hbm.at[b, 0, pl.ds(ci * ECH, ECH)],
                            ebuf_v.at[0])
            pltpu.sync_copy(edges_hbm.at[b, 1, pl.ds(ci * ECH, ECH)],
                            ebuf_v.at[1])

            def vec_body(i, c2):
                src = ebuf_v[0, pl.ds(i * 16, 16)]
                dst = ebuf_v[1, pl.ds(i * 16, 16)]
                pos = dst * N + src
                plsc.addupdate_scatter(acc_v, [pos], ones)
                return c2

            lax.fori_loop(0, ECH // 16, vec_body, 0)
            return carry

        lax.fori_loop(0, E // ECH, chunk_body, 0)
        pltpu.sync_copy(acc_v, hist_hbm.at[b])


@functools.cache
def _sc_gather_hist():
    mesh = plsc.VectorSubcoreMesh(
        core_axis_name="c", subcore_axis_name="s",
        num_cores=NC, num_subcores=NS)
    return pl.kernel(
        _sc_gather_hist_body,
        out_type=(
            jax.ShapeDtypeStruct((ROWS, D), jnp.float32),
            jax.ShapeDtypeStruct((B, N * N), jnp.float32),
        ),
        mesh=mesh,
        scratch_types=[
            pltpu.VMEM((GCH,), jnp.int32),
            pltpu.VMEM((GCH, D), jnp.float32),
            pltpu.VMEM((N * N,), jnp.float32),
            pltpu.VMEM((2, ECH), jnp.int32),
            pltpu.SemaphoreType.DMA,
        ],
        compiler_params=pltpu.CompilerParams(needs_layout_passes=False),
    )


def _tc_body(pm_ref, tok_ref, pos_ref, gam_ref, bet_ref, wg_ref, asrc_ref,
             adst_ref, hist_ref, cnt_ref, wout_ref, out_ref):
    f32 = jnp.float32
    hi = None
    tokb = tok_ref[0] + pos_ref[:]
    node = jnp.dot(pm_ref[0], tokb, preferred_element_type=f32,
                   precision=hi)
    mu = jnp.mean(node, axis=1, keepdims=True)
    xc = node - mu
    var = jnp.mean(xc * xc, axis=1, keepdims=True)
    node = xc * lax.rsqrt(var + 1e-12) * gam_ref[:] + bet_ref[:]
    wh = jnp.dot(node, wg_ref[:], preferred_element_type=f32,
                 precision=hi)
    s_src = lax.dot_general(asrc_ref[:], wh, (((1,), (1,)), ((), ())),
                            preferred_element_type=f32, precision=hi)
    s_dst = jnp.dot(wh, adst_ref[:], preferred_element_type=f32,
                    precision=hi)
    x = s_dst + s_src
    x = jnp.where(x >= 0, x, 0.2 * x)
    cmat = hist_ref[0]
    xm = jnp.where(cmat > 0, x, -1e30)
    emax = jnp.max(xm, axis=1, keepdims=True)
    emax = jnp.where(emax > -1e29, emax, 0.0)
    p = cmat * jnp.exp(xm - emax)
    den = jnp.sum(p, axis=1, keepdims=True)
    a = p / (den + 1e-16)
    msg = jnp.dot(a, wh, preferred_element_type=f32, precision=hi)
    g = jnp.where(msg > 0, msg, jnp.exp(msg) - 1.0)
    gs = jnp.sum(g, axis=0, keepdims=True)
    avg = gs / cnt_ref[pl.program_id(0), 0]
    out_ref[pl.ds(pl.program_id(0), 1), :] = jnp.dot(
        avg, wout_ref[:], preferred_element_type=f32, precision=hi)


_tc_call = pl.pallas_call(
    _tc_body,
    grid=(B,),
    in_specs=[
        pl.BlockSpec((1, N, T), lambda b: (b, 0, 0)),
        pl.BlockSpec((1, T, D), lambda b: (b, 0, 0)),
        pl.BlockSpec((T, D), lambda b: (0, 0)),
        pl.BlockSpec((1, D), lambda b: (0, 0)),
        pl.BlockSpec((1, D), lambda b: (0, 0)),
        pl.BlockSpec((D, D), lambda b: (0, 0)),
        pl.BlockSpec((1, D), lambda b: (0, 0)),
        pl.BlockSpec((D, 1), lambda b: (0, 0)),
        pl.BlockSpec((1, N, N), lambda b: (b, 0, 0)),
        pl.BlockSpec(memory_space=pltpu.SMEM),
        pl.BlockSpec((D, S), lambda b: (0, 0)),
    ],
    out_specs=pl.BlockSpec((B, S), lambda b: (0, 0)),
    out_shape=jax.ShapeDtypeStruct((B, S), jnp.float32),
    compiler_params=pltpu.CompilerParams(
        dimension_semantics=("arbitrary",)),
)


def kernel(input_ids, pooling_mask, edge_indices, node_counts,
           word_emb, pos_emb, ln_gamma, ln_beta, W_gat, a_src, a_dst, W_out):
    ids = input_ids.reshape(ROWS).astype(jnp.int32)
    edges = edge_indices.astype(jnp.int32)
    tok, hist = _sc_gather_hist()(ids, edges, word_emb)
    cnt = jnp.maximum(node_counts, 1).astype(jnp.float32).reshape(B, 1)
    logits = _tc_call(
        pooling_mask, tok.reshape(B, T, D), pos_emb,
        ln_gamma.reshape(1, D), ln_beta.reshape(1, D), W_gat,
        a_src.reshape(1, D), a_dst.reshape(D, 1),
        hist.reshape(B, N, N), cnt, W_out)
    return logits


# speedup vs baseline: 92.2908x; 1.4514x over previous
import functools

import jax
import jax.numpy as jnp
from jax import lax
from jax.experimental import pallas as pl
from jax.experimental.pallas import tpu as pltpu
from jax.experimental.pallas import tpu_sc as plsc

B, T, N, E, D, S = 8, 512, 256, 8192, 768, 3
NC, NS = 2, 16
NW = NC * NS
ROWS = B * T
HROWS = 32
GROWS = (ROWS - B * HROWS) // (NW - B)
RND = 32
EH = E // 2


def _row_dst(tok_hbm, g):
    return tok_hbm.at[g // T, pl.ds(g % T, RND), :]


def _sc_body(ids_hbm, edges_hbm, table_hbm, tok_hbm, hist_hbm,
             idx_v, rows_v, acc_v, ebuf_v, esem, gsem, wsema, wsemb):
    wid = lax.axis_index("s") * NC + lax.axis_index("c")
    buf_a = rows_v.at[pl.ds(0, RND)]
    buf_b = rows_v.at[pl.ds(RND, RND)]

    @pl.when(wid < B)
    def _():
        b = wid
        e0 = pltpu.async_copy(edges_hbm.at[b, 0, pl.ds(0, EH)],
                              ebuf_v.at[0], esem)
        e1 = pltpu.async_copy(edges_hbm.at[b, 1, pl.ds(0, EH)],
                              ebuf_v.at[1], esem)

        zv = jnp.zeros((16,), jnp.float32)

        def zero_body(i, carry):
            for j in range(32):
                acc_v[2 * i + j // 16, pl.ds((j % 16) * 16, 16)] = zv
            return carry

        lax.fori_loop(0, N // 2, zero_body, 0)

        ones = jnp.ones((16,), jnp.float32)

        def scan_body(i, carry):
            for u in range(4):
                k = (i * 4 + u) * 16
                src = ebuf_v[0, pl.ds(k, 16)]
                dst = ebuf_v[1, pl.ds(k, 16)]
                plsc.addupdate_scatter(acc_v, [dst, src], ones)
            return carry

        e0.wait()
        e1.wait()
        lax.fori_loop(0, EH // 64, scan_body, 0)
        e0b = pltpu.async_copy(edges_hbm.at[b, 0, pl.ds(EH, EH)],
                               ebuf_v.at[0], esem)
        e1b = pltpu.async_copy(edges_hbm.at[b, 1, pl.ds(EH, EH)],
                               ebuf_v.at[1], esem)
        e0b.wait()
        e1b.wait()
        lax.fori_loop(0, EH // 64, scan_body, 0)
        wh = pltpu.async_copy(acc_v, hist_hbm.at[b], wsemb)

        g = b * HROWS
        pltpu.sync_copy(ids_hbm.at[pl.ds(g, RND)], idx_v.at[pl.ds(0, RND)])
        pltpu.async_copy(table_hbm.at[idx_v.at[pl.ds(0, RND)]],
                         buf_a, gsem).wait()
        pltpu.sync_copy(buf_a, _row_dst(tok_hbm, g))
        wh.wait()

    @pl.when(wid >= B)
    def _():
        start = B * HROWS + (wid - B) * GROWS
        pltpu.sync_copy(ids_hbm.at[pl.ds(start, GROWS)], idx_v)

        def gat(r, buf):
            return pltpu.async_copy(
                table_hbm.at[idx_v.at[pl.ds(r * RND, RND)]], buf, gsem)

        def put(r, buf, sem):
            return pltpu.async_copy(buf, _row_dst(tok_hbm, start + r * RND),
                                    sem)

        bufs = (buf_a, buf_b)
        sems = (wsema, wsemb)
        nr = GROWS // RND
        g = gat(0, bufs[0])
        w_prev = None
        w_prev2 = None
        for r in range(nr):
            g.wait()
            if r + 1 < nr:
                if w_prev2 is not None:
                    w_prev2.wait()
                g = gat(r + 1, bufs[(r + 1) % 2])
            w = put(r, bufs[r % 2], sems[r % 2])
            w_prev2 = w_prev
            w_prev = w
        w_prev2.wait()
        w_prev.wait()


@functools.cache
def _sc_call():
    mesh = plsc.VectorSubcoreMesh(
        core_axis_name="c", subcore_axis_name="s",
        num_cores=NC, num_subcores=NS)
    return pl.kernel(
        _sc_body,
        out_type=(
            jax.ShapeDtypeStruct((B, T, D), jnp.float32),
            jax.ShapeDtypeStruct((B, N, N), jnp.float32),
        ),
        mesh=mesh,
        scratch_types=[
            pltpu.VMEM((GROWS,), jnp.int32),
            pltpu.VMEM((2 * RND, D), jnp.float32),
            pltpu.VMEM((N, N), jnp.float32),
            pltpu.VMEM((2, EH), jnp.int32),
            pltpu.SemaphoreType.DMA,
            pltpu.SemaphoreType.DMA,
            pltpu.SemaphoreType.DMA,
            pltpu.SemaphoreType.DMA,
        ],
        compiler_params=pltpu.CompilerParams(needs_layout_passes=False),
    )


def _tc_body(pm_ref, tok_ref, pos_ref, gam_ref, bet_ref, wg_ref, asrc_ref,
             adst_ref, hist_ref, cnt_ref, wout_ref, out_ref):
    f32 = jnp.float32
    tokb = tok_ref[0] + pos_ref[:]
    node = jnp.dot(pm_ref[0], tokb, preferred_element_type=f32)
    mu = jnp.mean(node, axis=1, keepdims=True)
    xc = node - mu
    var = jnp.mean(xc * xc, axis=1, keepdims=True)
    node = xc * lax.rsqrt(var + 1e-12) * gam_ref[:] + bet_ref[:]
    wh = jnp.dot(node, wg_ref[:], preferred_element_type=f32)
    s_src = lax.dot_general(asrc_ref[:], wh, (((1,), (1,)), ((), ())),
                            preferred_element_type=f32)
    s_dst = jnp.dot(wh, adst_ref[:], preferred_element_type=f32)
    x = s_dst + s_src
    x = jnp.where(x >= 0, x, 0.2 * x)
    cmat = hist_ref[0]
    xm = jnp.where(cmat > 0, x, -1e30)
    emax = jnp.max(xm, axis=1, keepdims=True)
    emax = jnp.where(emax > -1e29, emax, 0.0)
    p = cmat * jnp.exp(xm - emax)
    den = jnp.sum(p, axis=1, keepdims=True)
    a = p / (den + 1e-16)
    msg = jnp.dot(a, wh, preferred_element_type=f32)
    g = jnp.where(msg > 0, msg, jnp.exp(msg) - 1.0)
    gs = jnp.sum(g, axis=0, keepdims=True)
    avg = gs / cnt_ref[pl.program_id(0), 0]
    out_ref[pl.ds(pl.program_id(0), 1), :] = jnp.dot(
        avg, wout_ref[:], preferred_element_type=f32)


_tc_call = pl.pallas_call(
    _tc_body,
    grid=(B,),
    in_specs=[
        pl.BlockSpec((1, N, T), lambda b: (b, 0, 0)),
        pl.BlockSpec((1, T, D), lambda b: (b, 0, 0)),
        pl.BlockSpec((T, D), lambda b: (0, 0)),
        pl.BlockSpec((1, D), lambda b: (0, 0)),
        pl.BlockSpec((1, D), lambda b: (0, 0)),
        pl.BlockSpec((D, D), lambda b: (0, 0)),
        pl.BlockSpec((1, D), lambda b: (0, 0)),
        pl.BlockSpec((D, 1), lambda b: (0, 0)),
        pl.BlockSpec((1, N, N), lambda b: (b, 0, 0)),
        pl.BlockSpec(memory_space=pltpu.SMEM),
        pl.BlockSpec((D, S), lambda b: (0, 0)),
    ],
    out_specs=pl.BlockSpec((B, S), lambda b: (0, 0)),
    out_shape=jax.ShapeDtypeStruct((B, S), jnp.float32),
    compiler_params=pltpu.CompilerParams(
        dimension_semantics=("arbitrary",)),
)


def kernel(input_ids, pooling_mask, edge_indices, node_counts,
           word_emb, pos_emb, ln_gamma, ln_beta, W_gat, a_src, a_dst, W_out):
    ids = input_ids.reshape(ROWS).astype(jnp.int32)
    edges = edge_indices.astype(jnp.int32)
    tok, hist = _sc_call()(ids, edges, word_emb)
    cnt = jnp.maximum(node_counts, 1).astype(jnp.float32).reshape(B, 1)
    logits = _tc_call(
        pooling_mask, tok, pos_emb,
        ln_gamma.reshape(1, D), ln_beta.reshape(1, D), W_gat,
        a_src.reshape(1, D), a_dst.reshape(D, 1),
        hist, cnt, W_out)
    return logits


# speedup vs baseline: 94.5339x; 1.0243x over previous
import functools

import jax
import jax.numpy as jnp
from jax import lax
from jax.experimental import pallas as pl
from jax.experimental.pallas import tpu as pltpu
from jax.experimental.pallas import tpu_sc as plsc

B, T, N, E, D, S = 8, 512, 256, 8192, 768, 3
NC, NS = 2, 16
NW = NC * NS
ROWS = B * T
HROWS = 32
GROWS = (ROWS - B * HROWS) // (NW - B)
RND = 32
EH = E // 2


def _row_dst(tok_hbm, g):
    return tok_hbm.at[g // T, pl.ds(g % T, RND), :]


def _sc_body(ids_hbm, edges_hbm, table_hbm, tok_hbm, hist_hbm,
             idx_v, rows_v, acc_v, ebuf_v, esem, gsem, wsema, wsemb):
    wid = lax.axis_index("s") * NC + lax.axis_index("c")
    buf_a = rows_v.at[pl.ds(0, RND)]
    buf_b = rows_v.at[pl.ds(RND, RND)]

    @pl.when(wid < B)
    def _():
        b = wid
        e0 = pltpu.async_copy(edges_hbm.at[b, 0, pl.ds(0, EH)],
                              ebuf_v.at[0], esem)
        e1 = pltpu.async_copy(edges_hbm.at[b, 1, pl.ds(0, EH)],
                              ebuf_v.at[1], esem)

        zv = jnp.zeros((16,), jnp.float32)

        def zero_body(i, carry):
            for j in range(32):
                acc_v[2 * i + j // 16, pl.ds((j % 16) * 16, 16)] = zv
            return carry

        lax.fori_loop(0, N // 2, zero_body, 0)

        ones = jnp.ones((16,), jnp.float32)

        def scan_body(i, carry):
            for u in range(4):
                k = (i * 4 + u) * 16
                src = ebuf_v[0, pl.ds(k, 16)]
                dst = ebuf_v[1, pl.ds(k, 16)]
                plsc.addupdate_scatter(acc_v, [dst, src], ones)
            return carry

        e0.wait()
        e1.wait()
        lax.fori_loop(0, EH // 64, scan_body, 0)
        e0b = pltpu.async_copy(edges_hbm.at[b, 0, pl.ds(EH, EH)],
                               ebuf_v.at[0], esem)
        e1b = pltpu.async_copy(edges_hbm.at[b, 1, pl.ds(EH, EH)],
                               ebuf_v.at[1], esem)
        e0b.wait()
        e1b.wait()
        lax.fori_loop(0, EH // 64, scan_body, 0)
        wh = pltpu.async_copy(acc_v, hist_hbm.at[b], wsemb)

        g = b * HROWS
        pltpu.sync_copy(ids_hbm.at[pl.ds(g, RND)], idx_v.at[pl.ds(0, RND)])
        pltpu.async_copy(table_hbm.at[idx_v.at[pl.ds(0, RND)]],
                         buf_a, gsem).wait()
        pltpu.sync_copy(buf_a, _row_dst(tok_hbm, g))
        wh.wait()

    @pl.when(wid >= B)
    def _():
        start = B * HROWS + (wid - B) * GROWS
        pltpu.sync_copy(ids_hbm.at[pl.ds(start, GROWS)], idx_v)

        def gat(r, buf):
            return pltpu.async_copy(
                table_hbm.at[idx_v.at[pl.ds(r * RND, RND)]], buf, gsem)

        def put(r, buf, sem):
            return pltpu.async_copy(buf, _row_dst(tok_hbm, start + r * RND),
                                    sem)

        bufs = (buf_a, buf_b)
        sems = (wsema, wsemb)
        nr = GROWS // RND
        g = gat(0, bufs[0])
        w_prev = None
        w_prev2 = None
        for r in range(nr):
            g.wait()
            if r + 1 < nr:
                if w_prev2 is not None:
                    w_prev2.wait()
                g = gat(r + 1, bufs[(r + 1) % 2])
            w = put(r, bufs[r % 2], sems[r % 2])
            w_prev2 = w_prev
            w_prev = w
        w_prev2.wait()
        w_prev.wait()


@functools.cache
def _sc_call():
    mesh = plsc.VectorSubcoreMesh(
        core_axis_name="c", subcore_axis_name="s",
        num_cores=NC, num_subcores=NS)
    return pl.kernel(
        _sc_body,
        out_type=(
            jax.ShapeDtypeStruct((B, T, D), jnp.float32),
            jax.ShapeDtypeStruct((B, N, N), jnp.float32),
        ),
        mesh=mesh,
        scratch_types=[
            pltpu.VMEM((GROWS,), jnp.int32),
            pltpu.VMEM((2 * RND, D), jnp.float32),
            pltpu.VMEM((N, N), jnp.float32),
            pltpu.VMEM((2, EH), jnp.int32),
            pltpu.SemaphoreType.DMA,
            pltpu.SemaphoreType.DMA,
            pltpu.SemaphoreType.DMA,
            pltpu.SemaphoreType.DMA,
        ],
        compiler_params=pltpu.CompilerParams(needs_layout_passes=False),
    )


GPB = 2


def _tc_body(pm_ref, tok_ref, pos_ref, gam_ref, bet_ref, wg_ref, a2_ref,
             hist_ref, cnt_ref, wout_ref, out_ref):
    f32 = jnp.float32
    cs = lax.dot_general(wg_ref[:], a2_ref[:], (((1,), (1,)), ((), ())),
                         preferred_element_type=f32)
    for k in range(GPB):
        tokb = tok_ref[k] + pos_ref[:]
        node = jnp.dot(pm_ref[k], tokb, preferred_element_type=f32)
        mu = jnp.mean(node, axis=1, keepdims=True)
        xc = node - mu
        var = jnp.mean(xc * xc, axis=1, keepdims=True)
        node = xc * lax.rsqrt(var + 1e-12) * gam_ref[:] + bet_ref[:]
        wh = jnp.dot(node, wg_ref[:], preferred_element_type=f32)
        sv = jnp.dot(node, cs, preferred_element_type=f32)
        s_src = lax.dot_general(a2_ref[pl.ds(0, 1), :], wh,
                                (((1,), (1,)), ((), ())),
                                preferred_element_type=f32)
        s_dst = sv[:, 1:2]
        x = s_dst + s_src
        x = jnp.where(x >= 0, x, 0.2 * x)
        cmat = hist_ref[k]
        xm = jnp.where(cmat > 0, x, -1e30)
        emax = jnp.max(xm, axis=1, keepdims=True)
        emax = jnp.where(emax > -1e29, emax, 0.0)
        p = cmat * jnp.exp(xm - emax)
        den = jnp.sum(p, axis=1, keepdims=True)
        a = p / (den + 1e-16)
        msg = jnp.dot(a, wh, preferred_element_type=f32)
        g = jnp.where(msg > 0, msg, jnp.exp(msg) - 1.0)
        gs = jnp.sum(g, axis=0, keepdims=True)
        bidx = pl.program_id(0) * GPB + k
        cnt = jnp.maximum(cnt_ref[bidx], 1).astype(f32)
        avg = gs / cnt
        out_ref[pl.ds(bidx, 1), :] = jnp.dot(
            avg, wout_ref[:], preferred_element_type=f32)


_tc_call = pl.pallas_call(
    _tc_body,
    grid=(B // GPB,),
    in_specs=[
        pl.BlockSpec((GPB, N, T), lambda b: (b, 0, 0)),
        pl.BlockSpec((GPB, T, D), lambda b: (b, 0, 0)),
        pl.BlockSpec((T, D), lambda b: (0, 0)),
        pl.BlockSpec((1, D), lambda b: (0, 0)),
        pl.BlockSpec((1, D), lambda b: (0, 0)),
        pl.BlockSpec((D, D), lambda b: (0, 0)),
        pl.BlockSpec((2, D), lambda b: (0, 0)),
        pl.BlockSpec((GPB, N, N), lambda b: (b, 0, 0)),
        pl.BlockSpec(memory_space=pltpu.SMEM),
        pl.BlockSpec((D, S), lambda b: (0, 0)),
    ],
    out_specs=pl.BlockSpec((B, S), lambda b: (0, 0)),
    out_shape=jax.ShapeDtypeStruct((B, S), jnp.float32),
    compiler_params=pltpu.CompilerParams(
        dimension_semantics=("arbitrary",)),
)


def kernel(input_ids, pooling_mask, edge_indices, node_counts,
           word_emb, pos_emb, ln_gamma, ln_beta, W_gat, a_src, a_dst, W_out):
    ids = input_ids.reshape(ROWS).astype(jnp.int32)
    edges = edge_indices.astype(jnp.int32)
    tok, hist = _sc_call()(ids, edges, word_emb)
    a2 = jnp.stack([a_src, a_dst])
    logits = _tc_call(
        pooling_mask, tok, pos_emb,
        ln_gamma.reshape(1, D), ln_beta.reshape(1, D), W_gat,
        a2, hist, node_counts.astype(jnp.int32), W_out)
    return logits


# speedup vs baseline: 95.6778x; 1.0121x over previous
import functools

import jax
import jax.numpy as jnp
from jax import lax
from jax.experimental import pallas as pl
from jax.experimental.pallas import tpu as pltpu
from jax.experimental.pallas import tpu_sc as plsc

B, T, N, E, D, S = 8, 512, 256, 8192, 768, 3
NC, NS = 2, 16
NW = NC * NS
ROWS = B * T
HROWS = 32
GROWS = (ROWS - B * HROWS) // (NW - B)
RND = 16
NBUF = 3


def _row_dst(tok_hbm, g):
    return tok_hbm.at[g // T, pl.ds(g % T, RND), :]


def _sc_body(ids_hbm, edges_hbm, table_hbm, tok_hbm, hist_hbm,
             idx_v, rows_v, acc_v, ebuf_v, esem, gsem, wsema, wsemb, wsemc):
    wid = lax.axis_index("s") * NC + lax.axis_index("c")
    bufs = tuple(rows_v.at[pl.ds(k * RND, RND)] for k in range(NBUF))
    sems = (wsema, wsemb, wsemc)

    @pl.when(wid < B)
    def _():
        b = wid
        ed = pltpu.async_copy(edges_hbm.at[b], ebuf_v, esem)

        zv = jnp.zeros((16,), jnp.float32)

        def zero_body(i, carry):
            for j in range(32):
                acc_v[2 * i + j // 16, pl.ds((j % 16) * 16, 16)] = zv
            return carry

        lax.fori_loop(0, N // 2, zero_body, 0)

        ones = jnp.ones((16,), jnp.float32)

        def scan_body(i, carry):
            for u in range(4):
                k = (i * 4 + u) * 16
                src = ebuf_v[0, pl.ds(k, 16)]
                dst = ebuf_v[1, pl.ds(k, 16)]
                plsc.addupdate_scatter(acc_v, [dst, src], ones)
            return carry

        ed.wait()
        lax.fori_loop(0, E // 64, scan_body, 0)
        wh = pltpu.async_copy(acc_v, hist_hbm.at[b], wsemb)

        g = b * HROWS
        for r in range(HROWS // RND):
            pltpu.sync_copy(ids_hbm.at[pl.ds(g + r * RND, RND)],
                            idx_v.at[pl.ds(0, RND)])
            pltpu.async_copy(table_hbm.at[idx_v.at[pl.ds(0, RND)]],
                             bufs[0], gsem).wait()
            pltpu.sync_copy(bufs[0], _row_dst(tok_hbm, g + r * RND))
        wh.wait()

    @pl.when(wid >= B)
    def _():
        start = B * HROWS + (wid - B) * GROWS
        pltpu.sync_copy(ids_hbm.at[pl.ds(start, GROWS)], idx_v)

        def gat(r):
            return pltpu.async_copy(
                table_hbm.at[idx_v.at[pl.ds(r * RND, RND)]],
                bufs[r % NBUF], gsem)

        def put(r):
            return pltpu.async_copy(
                bufs[r % NBUF], _row_dst(tok_hbm, start + r * RND),
                sems[r % NBUF])

        nr = GROWS // RND
        gd = [gat(r) for r in range(NBUF - 1)]
        wd = [None] * nr
        for r in range(nr):
            gd[r].wait()
            nxt = r + NBUF - 1
            if nxt < nr:
                if wd and r >= 1 and nxt - NBUF >= 0:
                    wd[nxt - NBUF].wait()
                gd.append(gat(nxt))
            wd[r] = put(r)
        for r in range(nr - NBUF, nr):
            if r >= 0:
                wd[r].wait()


@functools.cache
def _sc_call():
    mesh = plsc.VectorSubcoreMesh(
        core_axis_name="c", subcore_axis_name="s",
        num_cores=NC, num_subcores=NS)
    return pl.kernel(
        _sc_body,
        out_type=(
            jax.ShapeDtypeStruct((B, T, D), jnp.float32),
            jax.ShapeDtypeStruct((B, N, N), jnp.float32),
        ),
        mesh=mesh,
        scratch_types=[
            pltpu.VMEM((GROWS,), jnp.int32),
            pltpu.VMEM((NBUF * RND, D), jnp.float32),
            pltpu.VMEM((N, N), jnp.float32),
            pltpu.VMEM((2, E), jnp.int32),
            pltpu.SemaphoreType.DMA,
            pltpu.SemaphoreType.DMA,
            pltpu.SemaphoreType.DMA,
            pltpu.SemaphoreType.DMA,
            pltpu.SemaphoreType.DMA,
        ],
        compiler_params=pltpu.CompilerParams(needs_layout_passes=False),
    )


GPB = 2


def _tc_body(pm_ref, tok_ref, pos_ref, gam_ref, bet_ref, wg_ref, a2_ref,
             hist_ref, cnt_ref, wout_ref, out_ref):
    f32 = jnp.float32
    cs = lax.dot_general(wg_ref[:], a2_ref[:], (((1,), (1,)), ((), ())),
                         preferred_element_type=f32)
    for k in range(GPB):
        tokb = tok_ref[k] + pos_ref[:]
        node = jnp.dot(pm_ref[k], tokb, preferred_element_type=f32)
        mu = jnp.mean(node, axis=1, keepdims=True)
        xc = node - mu
        var = jnp.mean(xc * xc, axis=1, keepdims=True)
        node = xc * lax.rsqrt(var + 1e-12) * gam_ref[:] + bet_ref[:]
        wh = jnp.dot(node, wg_ref[:], preferred_element_type=f32)
        sv = jnp.dot(node, cs, preferred_element_type=f32)
        s_src = lax.dot_general(a2_ref[pl.ds(0, 1), :], wh,
                                (((1,), (1,)), ((), ())),
                                preferred_element_type=f32)
        s_dst = sv[:, 1:2]
        x = s_dst + s_src
        x = jnp.where(x >= 0, x, 0.2 * x)
        cmat = hist_ref[k]
        xm = jnp.where(cmat > 0, x, -1e30)
        emax = jnp.max(xm, axis=1, keepdims=True)
        emax = jnp.where(emax > -1e29, emax, 0.0)
        p = cmat * jnp.exp(xm - emax)
        den = jnp.sum(p, axis=1, keepdims=True)
        a = p / (den + 1e-16)
        msg = jnp.dot(a, wh, preferred_element_type=f32)
        g = jnp.where(msg > 0, msg, jnp.exp(msg) - 1.0)
        gs = jnp.sum(g, axis=0, keepdims=True)
        bidx = pl.program_id(0) * GPB + k
        cnt = jnp.maximum(cnt_ref[bidx], 1).astype(f32)
        avg = gs / cnt
        out_ref[pl.ds(bidx, 1), :] = jnp.dot(
            avg, wout_ref[:], preferred_element_type=f32)


_tc_call = pl.pallas_call(
    _tc_body,
    grid=(B // GPB,),
    in_specs=[
        pl.BlockSpec((GPB, N, T), lambda b: (b, 0, 0)),
        pl.BlockSpec((GPB, T, D), lambda b: (b, 0, 0)),
        pl.BlockSpec((T, D), lambda b: (0, 0)),
        pl.BlockSpec((1, D), lambda b: (0, 0)),
        pl.BlockSpec((1, D), lambda b: (0, 0)),
        pl.BlockSpec((D, D), lambda b: (0, 0)),
        pl.BlockSpec((2, D), lambda b: (0, 0)),
        pl.BlockSpec((GPB, N, N), lambda b: (b, 0, 0)),
        pl.BlockSpec(memory_space=pltpu.SMEM),
        pl.BlockSpec((D, S), lambda b: (0, 0)),
    ],
    out_specs=pl.BlockSpec((B, S), lambda b: (0, 0)),
    out_shape=jax.ShapeDtypeStruct((B, S), jnp.float32),
    compiler_params=pltpu.CompilerParams(
        dimension_semantics=("arbitrary",)),
)


def kernel(input_ids, pooling_mask, edge_indices, node_counts,
           word_emb, pos_emb, ln_gamma, ln_beta, W_gat, a_src, a_dst, W_out):
    ids = input_ids.reshape(ROWS).astype(jnp.int32)
    edges = edge_indices.astype(jnp.int32)
    tok, hist = _sc_call()(ids, edges, word_emb)
    a2 = jnp.stack([a_src, a_dst])
    logits = _tc_call(
        pooling_mask, tok, pos_emb,
        ln_gamma.reshape(1, D), ln_beta.reshape(1, D), W_gat,
        a2, hist, node_counts.astype(jnp.int32), W_out)
    return logits
